# baseline (speedup 1.0000x reference)
import jax
import jax.numpy as jnp
from jax.experimental import pallas as pl
from jax.experimental.pallas import tpu as pltpu

N = 1024
IN_DIM = 128
HID = 256
LAYERS = 2
BLK = 256


def _glstm_kernel(x_ref, w_in_ref, b_in_ref, adj_ref, wx_ref, wh_ref, wm_ref,
                  bc_ref, wo_ref, bo_ref, out_ref, xh_ref):
    i = pl.program_id(0)

    @pl.when(i == 0)
    def _():
        xh_ref[...] = (
            jnp.dot(x_ref[...], w_in_ref[...], preferred_element_type=jnp.float32)
            + b_in_ref[...]
        )

    mask = adj_ref[...] != 0
    a = mask.astype(jnp.bfloat16)
    deg = jnp.sum(mask.astype(jnp.float32), axis=1, keepdims=True)
    xh_all = xh_ref[...].astype(jnp.bfloat16)
    agg = jnp.dot(a, xh_all, preferred_element_type=jnp.float32)
    agg = agg / jnp.maximum(deg, 1.0)

    xh = xh_ref[pl.ds(i * BLK, BLK), :]
    base = (
        jnp.dot(xh.astype(jnp.bfloat16), wx_ref[...],
                preferred_element_type=jnp.float32)
        + jnp.dot(agg.astype(jnp.bfloat16), wm_ref[...],
                  preferred_element_type=jnp.float32)
        + bc_ref[...]
    )

    h = xh
    c = xh
    for _ in range(LAYERS):
        gates = base + jnp.dot(h.astype(jnp.bfloat16), wh_ref[...],
                               preferred_element_type=jnp.float32)
        i_g = gates[:, 0 * HID:1 * HID]
        f_g = gates[:, 1 * HID:2 * HID]
        o_g = gates[:, 2 * HID:3 * HID]
        g_g = gates[:, 3 * HID:4 * HID]
        c = jax.nn.sigmoid(f_g) * c + jax.nn.sigmoid(i_g) * jnp.tanh(g_g)
        h = jax.nn.sigmoid(o_g) * jnp.tanh(c)

    out_ref[...] = (
        jnp.dot(h, wo_ref[...], preferred_element_type=jnp.float32) + bo_ref[...]
    )


def kernel(x, edge_indexs, edgenum, W_in, b_in, W_x, W_h, W_m, b_cell, W_out, b_out):
    x2 = x.reshape(N, IN_DIM)
    adj = edge_indexs.reshape(N, N)

    grid = N // BLK
    out = pl.pallas_call(
        _glstm_kernel,
        grid=(grid,),
        in_specs=[
            pl.BlockSpec((N, IN_DIM), lambda i: (0, 0)),
            pl.BlockSpec((IN_DIM, HID), lambda i: (0, 0)),
            pl.BlockSpec((1, HID), lambda i: (0, 0)),
            pl.BlockSpec((BLK, N), lambda i: (i, 0)),
            pl.BlockSpec((HID, 4 * HID), lambda i: (0, 0)),
            pl.BlockSpec((HID, 4 * HID), lambda i: (0, 0)),
            pl.BlockSpec((HID, 4 * HID), lambda i: (0, 0)),
            pl.BlockSpec((1, 4 * HID), lambda i: (0, 0)),
            pl.BlockSpec((HID, 1), lambda i: (0, 0)),
            pl.BlockSpec((1, 1), lambda i: (0, 0)),
        ],
        out_specs=pl.BlockSpec((BLK, 1), lambda i: (i, 0)),
        out_shape=jax.ShapeDtypeStruct((N, 1), jnp.float32),
        scratch_shapes=[pltpu.VMEM((N, HID), jnp.float32)],
    )(x2, W_in, b_in.reshape(1, HID), adj,
      W_x.astype(jnp.bfloat16), W_h.astype(jnp.bfloat16),
      W_m.astype(jnp.bfloat16),
      b_cell.reshape(1, 4 * HID), W_out, b_out.reshape(1, 1))

    return out.reshape(1, N, 1)


# speedup vs baseline: 1.4284x; 1.4284x over previous
import jax
import jax.numpy as jnp
from jax.experimental import pallas as pl
from jax.experimental.pallas import tpu as pltpu

N = 1024
IN_DIM = 128
HID = 256
LAYERS = 2
BLK = 256


def _glstm_kernel(x_ref, w_in_ref, b_in_ref, adj_ref, wx_ref, wh_ref, wm_ref,
                  bc_ref, wo_ref, bo_ref, out_ref, xh_ref):
    i = pl.program_id(0)

    @pl.when(i == 0)
    def _():
        xh_ref[...] = (
            jnp.dot(x_ref[...], w_in_ref[...], preferred_element_type=jnp.float32)
            + b_in_ref[...]
        )

    a = (adj_ref[...] != 0).astype(jnp.float32)
    deg = jnp.sum(a, axis=1, keepdims=True)
    agg = jnp.dot(a, xh_ref[...], preferred_element_type=jnp.float32)
    agg = agg / jnp.maximum(deg, 1.0)

    xh = xh_ref[pl.ds(i * BLK, BLK), :]
    base = (
        jnp.dot(xh, wx_ref[...], preferred_element_type=jnp.float32)
        + jnp.dot(agg, wm_ref[...], preferred_element_type=jnp.float32)
        + bc_ref[...]
    )

    h = xh
    c = xh
    for _ in range(LAYERS):
        gates = base + jnp.dot(h, wh_ref[...], preferred_element_type=jnp.float32)
        i_g = gates[:, 0 * HID:1 * HID]
        f_g = gates[:, 1 * HID:2 * HID]
        o_g = gates[:, 2 * HID:3 * HID]
        g_g = gates[:, 3 * HID:4 * HID]
        c = jax.nn.sigmoid(f_g) * c + jax.nn.sigmoid(i_g) * jnp.tanh(g_g)
        h = jax.nn.sigmoid(o_g) * jnp.tanh(c)

    out_ref[...] = (
        jnp.dot(h, wo_ref[...], preferred_element_type=jnp.float32) + bo_ref[...]
    )


def kernel(x, edge_indexs, edgenum, W_in, b_in, W_x, W_h, W_m, b_cell, W_out, b_out):
    x2 = x.reshape(N, IN_DIM)
    adj = edge_indexs.reshape(N, N)

    grid = N // BLK
    out = pl.pallas_call(
        _glstm_kernel,
        grid=(grid,),
        in_specs=[
            pl.BlockSpec((N, IN_DIM), lambda i: (0, 0)),
            pl.BlockSpec((IN_DIM, HID), lambda i: (0, 0)),
            pl.BlockSpec((1, HID), lambda i: (0, 0)),
            pl.BlockSpec((BLK, N), lambda i: (i, 0)),
            pl.BlockSpec((HID, 4 * HID), lambda i: (0, 0)),
            pl.BlockSpec((HID, 4 * HID), lambda i: (0, 0)),
            pl.BlockSpec((HID, 4 * HID), lambda i: (0, 0)),
            pl.BlockSpec((1, 4 * HID), lambda i: (0, 0)),
            pl.BlockSpec((HID, 1), lambda i: (0, 0)),
            pl.BlockSpec((1, 1), lambda i: (0, 0)),
        ],
        out_specs=pl.BlockSpec((BLK, 1), lambda i: (i, 0)),
        out_shape=jax.ShapeDtypeStruct((N, 1), jnp.float32),
        scratch_shapes=[pltpu.VMEM((N, HID), jnp.float32)],
    )(x2, W_in, b_in.reshape(1, HID), adj, W_x, W_h, W_m,
      b_cell.reshape(1, 4 * HID), W_out, b_out.reshape(1, 1))

    return out.reshape(1, N, 1)


# speedup vs baseline: 1.4513x; 1.0161x over previous
import jax
import jax.numpy as jnp
from jax.experimental import pallas as pl
from jax.experimental.pallas import tpu as pltpu

N = 1024
IN_DIM = 128
HID = 256
LAYERS = 2
BLK = 256


def _glstm_kernel(x_ref, w_in_ref, b_in_ref, adj_ref, wx_ref, wh_ref, wm_ref,
                  bc_ref, wo_ref, bo_ref, out_ref, xh_ref):
    i = pl.program_id(0)

    @pl.when(i == 0)
    def _():
        xh_ref[...] = (
            jnp.dot(x_ref[...], w_in_ref[...], preferred_element_type=jnp.float32)
            + b_in_ref[...]
        )

    a = (adj_ref[...] != 0).astype(jnp.float32)
    deg = jnp.sum(a, axis=1, keepdims=True)
    agg = jnp.dot(a, xh_ref[...], preferred_element_type=jnp.float32)
    agg = agg / jnp.maximum(deg, 1.0)

    xh = xh_ref[pl.ds(i * BLK, BLK), :]
    base = (
        jnp.dot(xh, wx_ref[...], preferred_element_type=jnp.float32)
        + jnp.dot(agg, wm_ref[...], preferred_element_type=jnp.float32)
        + bc_ref[...]
    )

    h = xh
    c = xh
    for _ in range(LAYERS):
        gates = base + jnp.dot(h, wh_ref[...], preferred_element_type=jnp.float32)
        i_g = gates[:, 0 * HID:1 * HID]
        f_g = gates[:, 1 * HID:2 * HID]
        o_g = gates[:, 2 * HID:3 * HID]
        g_g = gates[:, 3 * HID:4 * HID]
        sig_f = 0.5 * jnp.tanh(f_g * 0.5) + 0.5
        sig_i = 0.5 * jnp.tanh(i_g * 0.5) + 0.5
        sig_o = 0.5 * jnp.tanh(o_g * 0.5) + 0.5
        c = sig_f * c + sig_i * jnp.tanh(g_g)
        h = sig_o * jnp.tanh(c)

    out_ref[...] = (
        jnp.dot(h, wo_ref[...], preferred_element_type=jnp.float32) + bo_ref[...]
    )


def kernel(x, edge_indexs, edgenum, W_in, b_in, W_x, W_h, W_m, b_cell, W_out, b_out):
    x2 = x.reshape(N, IN_DIM)
    adj = edge_indexs.reshape(N, N)

    grid = N // BLK
    out = pl.pallas_call(
        _glstm_kernel,
        grid=(grid,),
        in_specs=[
            pl.BlockSpec((N, IN_DIM), lambda i: (0, 0)),
            pl.BlockSpec((IN_DIM, HID), lambda i: (0, 0)),
            pl.BlockSpec((1, HID), lambda i: (0, 0)),
            pl.BlockSpec((BLK, N), lambda i: (i, 0)),
            pl.BlockSpec((HID, 4 * HID), lambda i: (0, 0)),
            pl.BlockSpec((HID, 4 * HID), lambda i: (0, 0)),
            pl.BlockSpec((HID, 4 * HID), lambda i: (0, 0)),
            pl.BlockSpec((1, 4 * HID), lambda i: (0, 0)),
            pl.BlockSpec((HID, 1), lambda i: (0, 0)),
            pl.BlockSpec((1, 1), lambda i: (0, 0)),
        ],
        out_specs=pl.BlockSpec((BLK, 1), lambda i: (i, 0)),
        out_shape=jax.ShapeDtypeStruct((N, 1), jnp.float32),
        scratch_shapes=[pltpu.VMEM((N, HID), jnp.float32)],
    )(x2, W_in, b_in.reshape(1, HID), adj, W_x, W_h, W_m,
      b_cell.reshape(1, 4 * HID), W_out, b_out.reshape(1, 1))

    return out.reshape(1, N, 1)


# speedup vs baseline: 1.5615x; 1.0759x over previous
import jax
import jax.numpy as jnp
from jax.experimental import pallas as pl
from jax.experimental.pallas import tpu as pltpu

N = 1024
IN_DIM = 128
HID = 256
LAYERS = 2
BLK = 512


def _glstm_kernel(x_ref, w_in_ref, b_in_ref, adj_ref, wx_ref, wh_ref, wm_ref,
                  bc_ref, wo_ref, bo_ref, out_ref, xh_ref):
    i = pl.program_id(0)

    @pl.when(i == 0)
    def _():
        xh_ref[...] = (
            jnp.dot(x_ref[...], w_in_ref[...], preferred_element_type=jnp.float32)
            + b_in_ref[...]
        )

    a = (adj_ref[...] != 0).astype(jnp.float32)
    deg = jnp.sum(a, axis=1, keepdims=True)
    agg = jnp.dot(a, xh_ref[...], preferred_element_type=jnp.float32)
    agg = agg / jnp.maximum(deg, 1.0)

    xh = xh_ref[pl.ds(i * BLK, BLK), :]
    base = (
        jnp.dot(xh, wx_ref[...], preferred_element_type=jnp.float32)
        + jnp.dot(agg, wm_ref[...], preferred_element_type=jnp.float32)
        + bc_ref[...]
    )

    h = xh
    c = xh
    for _ in range(LAYERS):
        gates = base + jnp.dot(h, wh_ref[...], preferred_element_type=jnp.float32)
        i_g = gates[:, 0 * HID:1 * HID]
        f_g = gates[:, 1 * HID:2 * HID]
        o_g = gates[:, 2 * HID:3 * HID]
        g_g = gates[:, 3 * HID:4 * HID]
        sig_f = 0.5 * jnp.tanh(f_g * 0.5) + 0.5
        sig_i = 0.5 * jnp.tanh(i_g * 0.5) + 0.5
        sig_o = 0.5 * jnp.tanh(o_g * 0.5) + 0.5
        c = sig_f * c + sig_i * jnp.tanh(g_g)
        h = sig_o * jnp.tanh(c)

    out_ref[...] = (
        jnp.dot(h, wo_ref[...], preferred_element_type=jnp.float32) + bo_ref[...]
    )


def kernel(x, edge_indexs, edgenum, W_in, b_in, W_x, W_h, W_m, b_cell, W_out, b_out):
    x2 = x.reshape(N, IN_DIM)
    adj = edge_indexs.reshape(N, N)

    grid = N // BLK
    out = pl.pallas_call(
        _glstm_kernel,
        grid=(grid,),
        in_specs=[
            pl.BlockSpec((N, IN_DIM), lambda i: (0, 0)),
            pl.BlockSpec((IN_DIM, HID), lambda i: (0, 0)),
            pl.BlockSpec((1, HID), lambda i: (0, 0)),
            pl.BlockSpec((BLK, N), lambda i: (i, 0)),
            pl.BlockSpec((HID, 4 * HID), lambda i: (0, 0)),
            pl.BlockSpec((HID, 4 * HID), lambda i: (0, 0)),
            pl.BlockSpec((HID, 4 * HID), lambda i: (0, 0)),
            pl.BlockSpec((1, 4 * HID), lambda i: (0, 0)),
            pl.BlockSpec((HID, 1), lambda i: (0, 0)),
            pl.BlockSpec((1, 1), lambda i: (0, 0)),
        ],
        out_specs=pl.BlockSpec((BLK, 1), lambda i: (i, 0)),
        out_shape=jax.ShapeDtypeStruct((N, 1), jnp.float32),
        scratch_shapes=[pltpu.VMEM((N, HID), jnp.float32)],
    )(x2, W_in, b_in.reshape(1, HID), adj, W_x, W_h, W_m,
      b_cell.reshape(1, 4 * HID), W_out, b_out.reshape(1, 1))

    return out.reshape(1, N, 1)
